# baseline (speedup 1.0000x reference)
import functools

import jax
import jax.numpy as jnp
from jax import lax
from jax.experimental import pallas as pl
from jax.experimental.pallas import tpu as pltpu
from jax.experimental.pallas import tpu_sc as plsc

_C = 1000
_EPS = 1e-05
_CPAD = 1024
_NC, _NS, _L = 2, 16, 16


def _sc_body(n_rows, tgt_hbm, sw_hbm, tgt_cnt_v, ones_v, hist_v, w_v,
             tgt_out_v, out_v, shared_hist, shared_w):
    cid = lax.axis_index("c")
    sid = lax.axis_index("s")
    rows_cnt = tgt_cnt_v.shape[0]
    pltpu.sync_copy(tgt_hbm.at[pl.ds(sid * rows_cnt, rows_cnt)], tgt_cnt_v)

    def _fill16(ref, val, g):
        ref[pl.ds(g * _L, _L)] = jnp.full((_L,), val, jnp.float32)

    def _ones_loop(g, carry):
        _fill16(ones_v, 1.0, g)
        return carry

    lax.fori_loop(0, ones_v.shape[0] // _L, _ones_loop, 0)

    @pl.when(sid == 0)
    def _():
        def _zero_loop(g, carry):
            _fill16(hist_v, 0.0, g)
            return carry
        lax.fori_loop(0, _CPAD // _L, _zero_loop, 0)
        pltpu.sync_copy(hist_v, shared_hist)

    plsc.subcore_barrier()
    for j in range(rows_cnt):
        pltpu.sync_copy(ones_v, shared_hist.at[tgt_cnt_v.at[j]], add=True)
    plsc.subcore_barrier()

    @pl.when(sid == 0)
    def _():
        pltpu.sync_copy(shared_hist, hist_v)

        def _wloop(g, acc):
            cvec = hist_v[pl.ds(g * _L, _L)]
            idx = g * _L + lax.iota(jnp.int32, _L)
            wv = jnp.where(idx < _C, 1.0 / (cvec + _EPS), 0.0)
            w_v[pl.ds(g * _L, _L)] = wv
            return acc + wv

        acc = lax.fori_loop(0, _CPAD // _L, _wloop,
                            jnp.zeros((_L,), jnp.float32))
        total = acc[0]
        for k in range(1, _L):
            total = total + acc[k]
        scale = jnp.full((_L,), float(_C), jnp.float32) / total

        def _sloop(g, carry):
            w_v[pl.ds(g * _L, _L)] = w_v[pl.ds(g * _L, _L)] * scale
            return carry

        lax.fori_loop(0, _CPAD // _L, _sloop, 0)
        pltpu.sync_copy(w_v, shared_w)

    plsc.subcore_barrier()

    wid = cid * _NS + sid
    rows_out = tgt_out_v.shape[0]
    pltpu.sync_copy(tgt_hbm.at[pl.ds(wid * rows_out, rows_out)], tgt_out_v)
    for j in range(rows_out):
        pltpu.sync_copy(shared_w.at[tgt_out_v.at[j]], out_v.at[j])
    pltpu.sync_copy(out_v, sw_hbm.at[pl.ds(wid * rows_out, rows_out)])


def _sample_weights(targets):
    n = targets.shape[0]
    t2 = targets.astype(jnp.int32).reshape(n // 128, 128)
    rows_cnt = (n // 128) // _NS
    rows_out = (n // 128) // (_NC * _NS)
    mesh = plsc.VectorSubcoreMesh(core_axis_name="c", subcore_axis_name="s")
    sck = functools.partial(
        pl.kernel,
        out_type=jax.ShapeDtypeStruct((n // 128, 128), jnp.float32),
        mesh=mesh,
        scratch_types=[
            pltpu.VMEM((rows_cnt, 128), jnp.int32),
            pltpu.VMEM((128,), jnp.float32),
            pltpu.VMEM((_CPAD,), jnp.float32),
            pltpu.VMEM((_CPAD,), jnp.float32),
            pltpu.VMEM((rows_out, 128), jnp.int32),
            pltpu.VMEM((rows_out, 128), jnp.float32),
            pltpu.VMEM_SHARED((_CPAD,), jnp.float32),
            pltpu.VMEM_SHARED((_CPAD,), jnp.float32),
        ],
    )(functools.partial(_sc_body, n))
    return sck(t2)


def _tc_body(n_total, t_ref, sw_ref, x_ref, out_ref, acc_ref):
    i = pl.program_id(0)
    n = pl.num_programs(0)
    x = x_ref[...]
    t = t_ref[0, 0, :]
    sw = sw_ref[0, 0, :]
    m = jnp.max(x, axis=1, keepdims=True)
    e = jnp.exp(x - m)
    s = jnp.sum(e, axis=1)
    lse = m[:, 0] + jnp.log(s)
    cols = lax.broadcasted_iota(jnp.int32, x.shape, 1)
    picked = jnp.sum(jnp.where(cols == t[:, None], x, 0.0), axis=1)
    part = jnp.sum((lse - picked) * sw)

    @pl.when(i == 0)
    def _():
        acc_ref[0, 0] = part

    @pl.when(i > 0)
    def _():
        acc_ref[0, 0] = acc_ref[0, 0] + part

    @pl.when(i == n - 1)
    def _():
        out_ref[0, 0] = acc_ref[0, 0] / n_total


def kernel(inputs, targets):
    n_total, c = inputs.shape
    sw = _sample_weights(targets).reshape(n_total)
    block = 2048
    grid = n_total // block
    t3 = targets.astype(jnp.int32).reshape(grid, 1, block)
    sw3 = sw.reshape(grid, 1, block)
    body = functools.partial(_tc_body, float(n_total))
    out = pl.pallas_call(
        body,
        grid=(grid,),
        in_specs=[
            pl.BlockSpec((1, 1, block), lambda i: (i, 0, 0)),
            pl.BlockSpec((1, 1, block), lambda i: (i, 0, 0)),
            pl.BlockSpec((block, c), lambda i: (i, 0)),
        ],
        out_specs=pl.BlockSpec((1, 1), lambda i: (0, 0),
                               memory_space=pltpu.SMEM),
        out_shape=jax.ShapeDtypeStruct((1, 1), jnp.float32),
        scratch_shapes=[
            pltpu.SMEM((1, 1), jnp.float32),
        ],
    )(t3, sw3, inputs)
    return out[0, 0]
